# Initial kernel scaffold; baseline (speedup 1.0000x reference)
import jax
import jax.numpy as jnp
from jax.experimental import pallas as pl


def kernel(x, raw_embedding):
    raise NotImplementedError("write your pallas kernel here")


# speedup vs baseline: 1.0940x; 1.0940x over previous
import functools

import jax
import jax.numpy as jnp
from jax import lax
from jax.experimental import pallas as pl
from jax.experimental.pallas import tpu as pltpu
from jax.experimental.pallas import tpu_sc as plsc

_D = 64
_G = 128
_SCALE = 8.0


def _permute16(x, idx):
    dnums = lax.GatherDimensionNumbers(
        offset_dims=(), collapsed_slice_dims=(0,), start_index_map=(0,)
    )
    return lax.gather(
        x,
        idx[:, None],
        dimension_numbers=dnums,
        slice_sizes=(1,),
        mode=lax.GatherScatterMode.PROMISE_IN_BOUNDS,
    )


def _lane_sum16(x):
    i = lax.iota(jnp.int32, 16)
    for k in (8, 4, 2, 1):
        x = x + _permute16(x, i ^ k)
    return x


def _rsqrt16(s):
    xi = lax.bitcast_convert_type(s, jnp.int32)
    yi = jnp.int32(0x5F3759DF) - lax.shift_right_logical(xi, 1)
    y = lax.bitcast_convert_type(yi, jnp.float32)
    xh = s * 0.5
    for _ in range(3):
        y = y * (1.5 - xh * y * y)
    return y


@functools.partial(jax.jit, static_argnames=("n_rows", "per_w"))
def _lookup_normalize(idx_flat, table, *, n_rows, per_w):
    mesh = plsc.VectorSubcoreMesh(core_axis_name="c", subcore_axis_name="s")
    info = plsc.get_sparse_core_info()
    nc = info.num_cores
    n_groups = per_w // _G

    @functools.partial(
        pl.kernel,
        mesh=mesh,
        out_type=jax.ShapeDtypeStruct((n_rows, _D), jnp.float32),
        compiler_params=pltpu.CompilerParams(use_tc_tiling_on_sc=False),
        scratch_types=[
            pltpu.VMEM((per_w,), jnp.int32),
            pltpu.VMEM((_G, _D), jnp.float32),
            pltpu.SemaphoreType.DMA,
        ],
    )
    def body(idx_hbm, table_hbm, out_hbm, idx_v, buf_v, sem):
        wid = lax.axis_index("s") * nc + lax.axis_index("c")
        base = wid * per_w
        pltpu.sync_copy(idx_hbm.at[pl.ds(base, per_w)], idx_v)

        def group(g, carry):
            pltpu.async_copy(
                table_hbm.at[idx_v.at[pl.ds(g * _G, _G)]], buf_v, sem
            ).wait()

            def row(r, c):
                v0 = buf_v[r, pl.ds(0, 16)]
                v1 = buf_v[r, pl.ds(16, 16)]
                v2 = buf_v[r, pl.ds(32, 16)]
                v3 = buf_v[r, pl.ds(48, 16)]
                acc = v0 * v0 + v1 * v1 + v2 * v2 + v3 * v3
                ssq = jnp.maximum(_lane_sum16(acc), 1e-24)
                scale = _rsqrt16(ssq) * _SCALE
                buf_v[r, pl.ds(0, 16)] = v0 * scale
                buf_v[r, pl.ds(16, 16)] = v1 * scale
                buf_v[r, pl.ds(32, 16)] = v2 * scale
                buf_v[r, pl.ds(48, 16)] = v3 * scale
                return c

            lax.fori_loop(0, _G, row, 0)
            pltpu.sync_copy(buf_v, out_hbm.at[pl.ds(base + g * _G, _G)])
            return carry

        lax.fori_loop(0, n_groups, group, 0)

    return body(idx_flat, table)


def kernel(x, raw_embedding):
    b, s = x.shape
    n_rows = b * s
    info = plsc.get_sparse_core_info()
    nw = info.num_cores * info.num_subcores
    per_w = n_rows // nw
    assert per_w * nw == n_rows and per_w % _G == 0
    idx_flat = x.reshape(-1).astype(jnp.int32)
    out = _lookup_normalize(idx_flat, raw_embedding, n_rows=n_rows, per_w=per_w)
    return out.reshape(b, s, _D)


# speedup vs baseline: 1.8989x; 1.7357x over previous
import functools

import jax
import jax.numpy as jnp
from jax import lax
from jax.experimental import pallas as pl
from jax.experimental.pallas import tpu as pltpu
from jax.experimental.pallas import tpu_sc as plsc

_D = 64
_G = 128
_SCALE = 8.0


def _permute16(x, idx):
    dnums = lax.GatherDimensionNumbers(
        offset_dims=(), collapsed_slice_dims=(0,), start_index_map=(0,)
    )
    return lax.gather(
        x,
        idx[:, None],
        dimension_numbers=dnums,
        slice_sizes=(1,),
        mode=lax.GatherScatterMode.PROMISE_IN_BOUNDS,
    )


def _lane_sum16(x):
    i = lax.iota(jnp.int32, 16)
    for k in (8, 4, 2, 1):
        x = x + _permute16(x, i ^ k)
    return x


def _rsqrt16(s):
    xi = lax.bitcast_convert_type(s, jnp.int32)
    yi = jnp.int32(0x5F3759DF) - lax.shift_right_logical(xi, 1)
    y = lax.bitcast_convert_type(yi, jnp.float32)
    xh = s * 0.5
    for _ in range(2):
        y = y * (1.5 - xh * y * y)
    return y


@functools.partial(jax.jit, static_argnames=("n_rows", "per_w"))
def _lookup_normalize(idx_flat, table, *, n_rows, per_w):
    mesh = plsc.VectorSubcoreMesh(core_axis_name="c", subcore_axis_name="s")
    info = plsc.get_sparse_core_info()
    nc = info.num_cores
    n_groups = per_w // _G

    nbuf = 4
    assert n_groups % nbuf == 0 and n_groups >= 2 * nbuf

    @functools.partial(
        pl.kernel,
        mesh=mesh,
        out_type=jax.ShapeDtypeStruct((n_rows, _D), jnp.float32),
        compiler_params=pltpu.CompilerParams(use_tc_tiling_on_sc=False),
        scratch_types=[
            pltpu.VMEM((per_w,), jnp.int32),
            pltpu.VMEM((nbuf, _G, _D), jnp.float32),
            pltpu.SemaphoreType.DMA((nbuf,)),
            pltpu.SemaphoreType.DMA((nbuf,)),
        ],
    )
    def body(idx_hbm, table_hbm, out_hbm, idx_v, buf_v, gsem, ssem):
        wid = lax.axis_index("s") * nc + lax.axis_index("c")
        base = wid * per_w
        pltpu.sync_copy(idx_hbm.at[pl.ds(base, per_w)], idx_v)

        def gcopy(g, b):
            return pltpu.make_async_copy(
                table_hbm.at[idx_v.at[pl.ds(g * _G, _G)]],
                buf_v.at[b],
                gsem.at[b],
            )

        def scopy(g, b):
            return pltpu.make_async_copy(
                buf_v.at[b],
                out_hbm.at[pl.ds(base + g * _G, _G)],
                ssem.at[b],
            )

        def normalize_group(bref):
            def rows4(r4, c):
                for k in range(4):
                    r = r4 * 4 + k
                    v0 = bref[r, pl.ds(0, 16)]
                    v1 = bref[r, pl.ds(16, 16)]
                    v2 = bref[r, pl.ds(32, 16)]
                    v3 = bref[r, pl.ds(48, 16)]
                    acc = v0 * v0 + v1 * v1 + v2 * v2 + v3 * v3
                    ssq = jnp.maximum(_lane_sum16(acc), 1e-24)
                    scale = _rsqrt16(ssq) * _SCALE
                    bref[r, pl.ds(0, 16)] = v0 * scale
                    bref[r, pl.ds(16, 16)] = v1 * scale
                    bref[r, pl.ds(32, 16)] = v2 * scale
                    bref[r, pl.ds(48, 16)] = v3 * scale
                return c

            lax.fori_loop(0, _G // 4, rows4, 0)

        gcopy(0, 0).start()
        gcopy(1, 1).start()

        def outer(q, carry):
            for b in range(nbuf):
                g = q * nbuf + b
                nb = (b + 2) % nbuf

                @pl.when(g + 2 < n_groups)
                def _prefetch():
                    @pl.when(g >= 2)
                    def _drain_store():
                        scopy(g - 2, nb).wait()

                    gcopy(g + 2, nb).start()

                gcopy(g, b).wait()
                normalize_group(buf_v.at[b])
                scopy(g, b).start()
            return carry

        lax.fori_loop(0, n_groups // nbuf, outer, 0)
        scopy(n_groups - 2, (n_groups - 2) % nbuf).wait()
        scopy(n_groups - 1, (n_groups - 1) % nbuf).wait()

    return body(idx_flat, table)


def kernel(x, raw_embedding):
    b, s = x.shape
    n_rows = b * s
    info = plsc.get_sparse_core_info()
    nw = info.num_cores * info.num_subcores
    per_w = n_rows // nw
    assert per_w * nw == n_rows and per_w % _G == 0
    idx_flat = x.reshape(-1).astype(jnp.int32)
    out = _lookup_normalize(idx_flat, raw_embedding, n_rows=n_rows, per_w=per_w)
    return out.reshape(b, s, _D)
